# baseline (speedup 1.0000x reference)
import jax
import jax.numpy as jnp
from jax import lax
from jax.experimental import pallas as pl
from jax.experimental.pallas import tpu as pltpu
from jax.experimental.pallas import tpu_sc as plsc

B = 1024
L_DOC = 200
L_Q = 50
LT = L_DOC + L_Q
D = 64
DP = 128
VOCAB = 1000000
NW = 32
NSLAB = -(-VOCAB // DP)
VPAD = NSLAB * DP
K = 4
NG = NSLAB // K
RING_I = 60
TPW = B * LT // NW
CH = 80
NCH = TPW // CH
NBUF = 2
PEXT = 320


def _tr_kernel(tab_hbm, tail_hbm, scr_hbm, in_v, out_v, *sems):
    isems = sems[:NBUF]
    osems = sems[NBUF:]
    wid = lax.axis_index("s") * 2 + lax.axis_index("c")

    def in_copies(g, b):
        return [pltpu.make_async_copy(
            tab_hbm.at[pl.ds(fb * 8, 8), pl.ds(g * K * DP, K * DP)],
            in_v.at[b, pl.ds(fb * 8, 8)], isems[b]) for fb in range(8)]

    def out_copy(s, ob):
        return pltpu.make_async_copy(
            out_v.at[ob], scr_hbm.at[pl.ds(s * DP, DP)], osems[ob])

    rows_j = [lax.iota(jnp.int32, 16) + j * 16 for j in range(4)]

    def transpose(b, k, ob):
        @plsc.parallel_loop(0, DP, step=8, unroll=4)
        def tr_body(l0):
            for dl in range(8):
                l = l0 + dl
                colv = jnp.full((16,), k * DP + l, jnp.int32)
                for j in range(4):
                    out_v[ob, l, pl.ds(j * 16, 16)] = plsc.load_gather(
                        in_v.at[b], [rows_j[j], colv])

    def do_group(g, b, first):
        for k in range(K):
            ob = k % 2
            if k >= 2:
                out_copy(g * K + k - 2, ob).wait()
            else:
                @pl.when(jnp.logical_not(first))
                def _wait_out():
                    out_copy((g - NW) * K + k + 2, ob).wait()
            transpose(b, k, ob)
            out_copy(g * K + k, ob).start()

    for b in range(NBUF):
        for cp in in_copies(wid + b * NW, b):
            cp.start()

    def louter(i0, carry):
        for b in range(NBUF):
            i = i0 * NBUF + b
            g = wid + i * NW
            for cp in in_copies(g, b):
                cp.wait()
            do_group(g, b, i == 0)

            @pl.when(i + NBUF < RING_I)
            def _next_in():
                for cp in in_copies(g + NBUF * NW, b):
                    cp.start()
        return carry

    lax.fori_loop(0, RING_I // NBUF, louter, 0)

    g_last = wid + (RING_I - 1) * NW
    out_copy(g_last * K + 2, 0).wait()
    out_copy(g_last * K + 3, 1).wait()

    for g_t in (wid + RING_I * NW,):
        @pl.when(g_t < NG)
        def _tail_group():
            for cp in in_copies(g_t, 0):
                cp.start()
            for cp in in_copies(g_t, 0):
                cp.wait()
            for k in range(K):
                transpose(0, k, 0)
                out_copy(g_t * K + k, 0).start()
                out_copy(g_t * K + k, 0).wait()

    @pl.when(wid == 0)
    def _tail_group2():
        g_t = NG - 1
        for cp in in_copies(g_t, 0):
            cp.start()
        for cp in in_copies(g_t, 0):
            cp.wait()
        for k in range(K):
            transpose(0, k, 0)
            out_copy(g_t * K + k, 0).start()
            out_copy(g_t * K + k, 0).wait()

    @pl.when(wid == 1)
    def _tail_slab():
        pltpu.sync_copy(tail_hbm, scr_hbm.at[pl.ds(NG * K * DP, D)])


def _emb_kernel(ids_hbm, pos_hbm, scr_hbm, out_hbm,
                idx_v, pos_v, rows_g, rows_o, *sems):
    gsems = sems[:NBUF]
    osems = sems[NBUF:]
    wid = lax.axis_index("s") * 2 + lax.axis_index("c")
    base = wid * TPW
    pltpu.sync_copy(ids_hbm.at[wid], idx_v)
    pltpu.sync_copy(pos_hbm, pos_v)

    def gather(c, b):
        return pltpu.make_async_copy(
            scr_hbm.at[idx_v.at[c]], rows_g.at[b], gsems[b])

    def out_copy(c, b):
        return pltpu.make_async_copy(
            rows_o.at[b], out_hbm.at[pl.ds(base + c * CH, CH)], osems[b])

    for b in range(NBUF):
        gather(b, b).start()

    def outer(c0, carry):
        for b in range(NBUF):
            c = c0 * NBUF + b
            gather(c, b).wait()

            @pl.when(c0 > 0)
            def _wait_prev():
                out_copy(c - NBUF, b).wait()

            poff = lax.rem(c * CH, LT)

            def add_body(r, carry2):
                for j in range(4):
                    sl = pl.ds(j * 16, 16)
                    rows_o[b, r, sl] = rows_g[b, r, sl] + pos_v[poff + r, sl]
                return carry2

            lax.fori_loop(0, CH, add_body, 0)

            @pl.when(c + NBUF < NCH)
            def _next_gather():
                gather(c + NBUF, b).start()

            out_copy(c, b).start()
        return carry

    lax.fori_loop(0, NCH // NBUF, outer, 0)
    for b in range(NBUF):
        out_copy(NCH - NBUF + b, b).wait()


def kernel(x1_ids, x2_ids, emb_table, pos_table):
    ids = jnp.concatenate([x1_ids, x2_ids], axis=1).astype(jnp.int32)
    ids_r = ids.reshape(NW, NCH, CH)
    pos_full = jnp.concatenate([pos_table[:L_DOC], pos_table[:L_Q]], axis=0)
    pos_ext = jnp.concatenate([pos_full, pos_full[: PEXT - LT]], axis=0)

    emb_tail128 = jnp.pad(emb_table[NG * K * DP:], ((0, 0), (0, DP - D)))
    mesh = plsc.VectorSubcoreMesh(core_axis_name="c", subcore_axis_name="s")
    params = pltpu.CompilerParams(use_tc_tiling_on_sc=True, needs_layout_passes=False, disable_bounds_checks=True)

    scr = pl.kernel(
        _tr_kernel,
        out_type=jax.ShapeDtypeStruct((VPAD, DP), jnp.float32),
        mesh=mesh,
        compiler_params=params,
        scratch_types=[
            pltpu.VMEM((NBUF, D, K * DP), jnp.float32),
            pltpu.VMEM((2, DP, DP), jnp.float32),
        ] + [pltpu.SemaphoreType.DMA] * (2 * NBUF),
    )(emb_table.T, emb_tail128)

    out = pl.kernel(
        _emb_kernel,
        out_type=jax.ShapeDtypeStruct((B * LT, D), jnp.float32),
        mesh=mesh,
        compiler_params=params,
        scratch_types=[
            pltpu.VMEM((NCH, CH), jnp.int32),
            pltpu.VMEM((PEXT, D), jnp.float32),
            pltpu.VMEM((NBUF, CH, DP), jnp.float32),
            pltpu.VMEM((NBUF, CH, D), jnp.float32),
        ] + [pltpu.SemaphoreType.DMA] * (2 * NBUF),
    )(ids_r, pos_ext, scr)
    return out.reshape(B, LT, D)


# speedup vs baseline: 1.0111x; 1.0111x over previous
import jax
import jax.numpy as jnp
from jax import lax
from jax.experimental import pallas as pl
from jax.experimental.pallas import tpu as pltpu
from jax.experimental.pallas import tpu_sc as plsc

B = 1024
L_DOC = 200
L_Q = 50
LT = L_DOC + L_Q
D = 64
DP = 128
VOCAB = 1000000
NW = 32
NSLAB = -(-VOCAB // DP)
VPAD = NSLAB * DP
K = 4
NG = NSLAB // K
RING_I = 60
TPW = B * LT // NW
CH = 80
NCH = TPW // CH
NBUF = 2
PEXT = 320


def _tr_kernel(tab_hbm, tail_hbm, scr_hbm, in_v, out_v, *sems):
    isems = sems[:NBUF]
    osems = sems[NBUF:]
    wid = lax.axis_index("s") * 2 + lax.axis_index("c")

    def in_copies(g, b):
        return [pltpu.make_async_copy(
            tab_hbm.at[pl.ds(fb * 8, 8), pl.ds(g * K * DP, K * DP)],
            in_v.at[b, pl.ds(fb * 8, 8)], isems[b]) for fb in range(8)]

    def out_copy(s, ob):
        return pltpu.make_async_copy(
            out_v.at[ob], scr_hbm.at[pl.ds(s * DP, DP)], osems[ob])

    rows_j = [lax.iota(jnp.int32, 16) + j * 16 for j in range(4)]

    def transpose(b, k, ob):
        base = jnp.full((16,), k * DP, jnp.int32)

        @plsc.parallel_loop(0, DP, step=1, unroll=8, carry=base)
        def tr_body(l, colv):
            for j in range(4):
                out_v[ob, l, pl.ds(j * 16, 16)] = plsc.load_gather(
                    in_v.at[b], [rows_j[j], colv])
            return colv + 1

    def do_group(g, b, first):
        for k in range(K):
            ob = k % 2
            if k >= 2:
                out_copy(g * K + k - 2, ob).wait()
            else:
                @pl.when(jnp.logical_not(first))
                def _wait_out():
                    out_copy((g - NW) * K + k + 2, ob).wait()
            transpose(b, k, ob)
            out_copy(g * K + k, ob).start()

    for b in range(NBUF):
        for cp in in_copies(wid + b * NW, b):
            cp.start()

    def louter(i0, carry):
        for b in range(NBUF):
            i = i0 * NBUF + b
            g = wid + i * NW
            for cp in in_copies(g, b):
                cp.wait()
            do_group(g, b, i == 0)

            @pl.when(i + NBUF < RING_I)
            def _next_in():
                for cp in in_copies(g + NBUF * NW, b):
                    cp.start()
        return carry

    lax.fori_loop(0, RING_I // NBUF, louter, 0)

    g_last = wid + (RING_I - 1) * NW
    out_copy(g_last * K + 2, 0).wait()
    out_copy(g_last * K + 3, 1).wait()

    for g_t in (wid + RING_I * NW,):
        @pl.when(g_t < NG)
        def _tail_group():
            for cp in in_copies(g_t, 0):
                cp.start()
            for cp in in_copies(g_t, 0):
                cp.wait()
            for k in range(K):
                transpose(0, k, 0)
                out_copy(g_t * K + k, 0).start()
                out_copy(g_t * K + k, 0).wait()

    @pl.when(wid == 0)
    def _tail_group2():
        g_t = NG - 1
        for cp in in_copies(g_t, 0):
            cp.start()
        for cp in in_copies(g_t, 0):
            cp.wait()
        for k in range(K):
            transpose(0, k, 0)
            out_copy(g_t * K + k, 0).start()
            out_copy(g_t * K + k, 0).wait()

    @pl.when(wid == 1)
    def _tail_slab():
        pltpu.sync_copy(tail_hbm, scr_hbm.at[pl.ds(NG * K * DP, D)])


def _emb_kernel(ids_hbm, pos_hbm, scr_hbm, out_hbm,
                idx_v, pos_v, rows_g, rows_o, *sems):
    gsems = sems[:NBUF]
    osems = sems[NBUF:]
    wid = lax.axis_index("s") * 2 + lax.axis_index("c")
    base = wid * TPW
    pltpu.sync_copy(ids_hbm.at[wid], idx_v)
    pltpu.sync_copy(pos_hbm, pos_v)

    def gather(c, b):
        return pltpu.make_async_copy(
            scr_hbm.at[idx_v.at[c]], rows_g.at[b], gsems[b])

    def out_copy(c, b):
        return pltpu.make_async_copy(
            rows_o.at[b], out_hbm.at[pl.ds(base + c * CH, CH)], osems[b])

    for b in range(NBUF):
        gather(b, b).start()

    def outer(c0, carry):
        for b in range(NBUF):
            c = c0 * NBUF + b
            gather(c, b).wait()

            @pl.when(c0 > 0)
            def _wait_prev():
                out_copy(c - NBUF, b).wait()

            poff = lax.rem(c * CH, LT)

            def add_body(r, carry2):
                for j in range(4):
                    sl = pl.ds(j * 16, 16)
                    rows_o[b, r, sl] = rows_g[b, r, sl] + pos_v[poff + r, sl]
                return carry2

            lax.fori_loop(0, CH, add_body, 0)

            @pl.when(c + NBUF < NCH)
            def _next_gather():
                gather(c + NBUF, b).start()

            out_copy(c, b).start()
        return carry

    lax.fori_loop(0, NCH // NBUF, outer, 0)
    for b in range(NBUF):
        out_copy(NCH - NBUF + b, b).wait()


def kernel(x1_ids, x2_ids, emb_table, pos_table):
    ids = jnp.concatenate([x1_ids, x2_ids], axis=1).astype(jnp.int32)
    ids_r = ids.reshape(NW, NCH, CH)
    pos_full = jnp.concatenate([pos_table[:L_DOC], pos_table[:L_Q]], axis=0)
    pos_ext = jnp.concatenate([pos_full, pos_full[: PEXT - LT]], axis=0)

    emb_tail128 = jnp.pad(emb_table[NG * K * DP:], ((0, 0), (0, DP - D)))
    mesh = plsc.VectorSubcoreMesh(core_axis_name="c", subcore_axis_name="s")
    params = pltpu.CompilerParams(use_tc_tiling_on_sc=True, needs_layout_passes=False, disable_bounds_checks=True)

    scr = pl.kernel(
        _tr_kernel,
        out_type=jax.ShapeDtypeStruct((VPAD, DP), jnp.float32),
        mesh=mesh,
        compiler_params=params,
        scratch_types=[
            pltpu.VMEM((NBUF, D, K * DP), jnp.float32),
            pltpu.VMEM((2, DP, DP), jnp.float32),
        ] + [pltpu.SemaphoreType.DMA] * (2 * NBUF),
    )(emb_table.T, emb_tail128)

    out = pl.kernel(
        _emb_kernel,
        out_type=jax.ShapeDtypeStruct((B * LT, D), jnp.float32),
        mesh=mesh,
        compiler_params=params,
        scratch_types=[
            pltpu.VMEM((NCH, CH), jnp.int32),
            pltpu.VMEM((PEXT, D), jnp.float32),
            pltpu.VMEM((NBUF, CH, DP), jnp.float32),
            pltpu.VMEM((NBUF, CH, D), jnp.float32),
        ] + [pltpu.SemaphoreType.DMA] * (2 * NBUF),
    )(ids_r, pos_ext, scr)
    return out.reshape(B, LT, D)


# speedup vs baseline: 1.2755x; 1.2615x over previous
import jax
import jax.numpy as jnp
from jax import lax
from jax.experimental import pallas as pl
from jax.experimental.pallas import tpu as pltpu
from jax.experimental.pallas import tpu_sc as plsc

B = 1024
L_DOC = 200
L_Q = 50
LT = L_DOC + L_Q
D = 64
NW = 32
TPW = B * LT // NW
CH = 80
NCH = TPW // CH
NBUF = 4
PEXT = CH * ((LT + CH - 1) // CH) + CH


def _emb_kernel(ids_hbm, pos_hbm, table_hbm, out_hbm,
                idx_v, pos_v, rows_g, rows_o, *sems):
    gsems = sems[:NBUF]
    osems = sems[NBUF:]
    wid = lax.axis_index("s") * 2 + lax.axis_index("c")
    base = wid * TPW
    pltpu.sync_copy(ids_hbm.at[wid], idx_v)
    pltpu.sync_copy(pos_hbm, pos_v)

    def gather(c, b):
        return pltpu.make_async_copy(
            table_hbm.at[idx_v.at[c]], rows_g.at[b], gsems[b])

    def out_copy(c, b):
        return pltpu.make_async_copy(
            rows_o.at[b], out_hbm.at[pl.ds(base + c * CH, CH)], osems[b])

    for b in range(NBUF):
        gather(b, b).start()

    def outer(c0, carry):
        for b in range(NBUF):
            c = c0 * NBUF + b
            gather(c, b).wait()

            @pl.when(c0 > 0)
            def _wait_prev():
                out_copy(c - NBUF, b).wait()

            poff = lax.rem(c * CH, LT)

            def add_body(r, carry2):
                for j in range(4):
                    sl = pl.ds(j * 16, 16)
                    rows_o[b, r, sl] = rows_g[b, r, sl] + pos_v[poff + r, sl]
                return carry2

            lax.fori_loop(0, CH, add_body, 0)

            @pl.when(c + NBUF < NCH)
            def _next_gather():
                gather(c + NBUF, b).start()

            out_copy(c, b).start()
        return carry

    lax.fori_loop(0, NCH // NBUF, outer, 0)
    for b in range(NBUF):
        out_copy(NCH - NBUF + b, b).wait()


def kernel(x1_ids, x2_ids, emb_table, pos_table):
    ids = jnp.concatenate([x1_ids, x2_ids], axis=1).astype(jnp.int32)
    ids_r = ids.reshape(NW, NCH, CH)
    pos_full = jnp.concatenate([pos_table[:L_DOC], pos_table[:L_Q]], axis=0)
    pos_ext = jnp.concatenate([pos_full, pos_full[: PEXT - LT]], axis=0)
    out = pl.kernel(
        _emb_kernel,
        out_type=jax.ShapeDtypeStruct((B * LT, D), jnp.float32),
        mesh=plsc.VectorSubcoreMesh(core_axis_name="c", subcore_axis_name="s"),
        compiler_params=pltpu.CompilerParams(use_tc_tiling_on_sc=False),
        scratch_types=[
            pltpu.VMEM((NCH, CH), jnp.int32),
            pltpu.VMEM((PEXT, D), jnp.float32),
            pltpu.VMEM((NBUF, CH, D), jnp.float32),
            pltpu.VMEM((NBUF, CH, D), jnp.float32),
        ] + [pltpu.SemaphoreType.DMA] * (2 * NBUF),
    )(ids_r, pos_ext, emb_table)
    return out.reshape(B, LT, D)
